# baseline (speedup 1.0000x reference)
import functools

import jax
import jax.numpy as jnp
from jax import lax
from jax.experimental import pallas as pl
from jax.experimental.pallas import tpu as pltpu
from jax.experimental.pallas import tpu_sc as plsc

NUM_CORES = 2
NUM_SUBCORES = 16
NUM_WORKERS = NUM_CORES * NUM_SUBCORES
LANES = 128
RCHUNK = 8
NBUF = 4
LOOKAHEAD = 2
IDX_BLOCKS = 4


@functools.lru_cache(maxsize=None)
def _make_gather(num_rows, dim, b0, b1):
    assert b0 % (NUM_WORKERS * RCHUNK) == 0
    rows_w = b0 // NUM_WORKERS
    n_chunks = rows_w // RCHUNK
    assert n_chunks % NBUF == 0 and n_chunks >= 2 * NBUF
    mesh = plsc.VectorSubcoreMesh(core_axis_name="c", subcore_axis_name="s")

    @functools.partial(
        pl.kernel,
        mesh=mesh,
        out_type=jax.ShapeDtypeStruct((b0, b1, dim), jnp.float32),
        compiler_params=pltpu.CompilerParams(use_tc_tiling_on_sc=False),
        scratch_types=[
            pltpu.VMEM((rows_w, b1), jnp.int32),
            pltpu.VMEM((rows_w // IDX_BLOCKS, LANES), jnp.int32),
            pltpu.VMEM((NBUF, RCHUNK, b1, dim), jnp.float32),
            pltpu.SemaphoreType.DMA((NBUF,)),
            pltpu.SemaphoreType.DMA((NBUF,)),
        ],
    )
    def gather_kernel(idx_hbm, table_hbm, out_hbm, idx_v, idx_raw, bufs,
                      gsem, osem):
        wid = lax.axis_index("s") * NUM_CORES + lax.axis_index("c")
        base = wid * rows_w
        br = rows_w // IDX_BLOCKS
        for blk in range(IDX_BLOCKS):
            pltpu.sync_copy(idx_hbm.at[pl.ds(base + blk * br, br)], idx_raw)

            def compact(r, carry):
                row = blk * br + r
                idx_v[row, pl.ds(0, 16)] = idx_raw[r, pl.ds(0, 16)]
                idx_v[row, pl.ds(b1 - 16, 16)] = idx_raw[r, pl.ds(b1 - 16, 16)]
                return carry

            lax.fori_loop(0, br, compact, 0)

        def gather_start(g, slot):
            for r in range(RCHUNK):
                pltpu.async_copy(
                    table_hbm.at[idx_v.at[g * RCHUNK + r]],
                    bufs.at[slot].at[r], gsem.at[slot])

        def gather_wait(g, slot):
            for r in range(RCHUNK):
                pltpu.make_async_copy(
                    table_hbm.at[idx_v.at[g * RCHUNK + r]],
                    bufs.at[slot].at[r], gsem.at[slot]).wait()

        def out_start(g, slot):
            pltpu.async_copy(bufs.at[slot],
                             out_hbm.at[pl.ds(base + g * RCHUNK, RCHUNK)],
                             osem.at[slot])

        def out_wait(g, slot):
            pltpu.make_async_copy(bufs.at[slot],
                                  out_hbm.at[pl.ds(base + g * RCHUNK, RCHUNK)],
                                  osem.at[slot]).wait()

        for s in range(LOOKAHEAD):
            gather_start(s, s)
        for g in range(NBUF - LOOKAHEAD):
            gather_start(g + LOOKAHEAD, g + LOOKAHEAD)
            gather_wait(g, g)
            out_start(g, g)

        start = NBUF - LOOKAHEAD
        n_main = n_chunks - NBUF

        def body(t, carry):
            for s in range(NBUF):
                g = start + t * NBUF + s
                slot = (start + s) % NBUF
                f = g + LOOKAHEAD
                fslot = (slot + LOOKAHEAD) % NBUF
                out_wait(f - NBUF, fslot)
                gather_start(f, fslot)
                gather_wait(g, slot)
                out_start(g, slot)
            return carry

        lax.fori_loop(0, n_main // NBUF, body, 0)

        for k in range(LOOKAHEAD):
            g = n_chunks - LOOKAHEAD + k
            slot = g % NBUF
            gather_wait(g, slot)
            out_start(g, slot)
        for k in range(NBUF):
            g = n_chunks - NBUF + k
            out_wait(g, g % NBUF)

    return gather_kernel


def kernel(input, weight):
    b0, b1 = input.shape
    num_rows, dim = weight.shape
    idx = input.astype(jnp.int32)
    idx = jnp.pad(idx, ((0, 0), (0, LANES - b1)))
    return _make_gather(num_rows, dim, b0, b1)(idx, weight)


# speedup vs baseline: 1.0059x; 1.0059x over previous
import functools

import jax
import jax.numpy as jnp
from jax import lax
from jax.experimental import pallas as pl
from jax.experimental.pallas import tpu as pltpu
from jax.experimental.pallas import tpu_sc as plsc

NUM_CORES = 2
NUM_SUBCORES = 16
NUM_WORKERS = NUM_CORES * NUM_SUBCORES
CHUNK = 128
NBUF = 8
LOOKAHEAD = 4


@functools.lru_cache(maxsize=None)
def _make_gather(num_rows, dim, batch):
    assert batch % (NUM_WORKERS * CHUNK) == 0
    n_chunks = batch // (NUM_WORKERS * CHUNK)
    assert n_chunks % NBUF == 0 and n_chunks >= 2 * NBUF
    rows_per_worker = n_chunks * CHUNK
    mesh = plsc.VectorSubcoreMesh(core_axis_name="c", subcore_axis_name="s")

    @functools.partial(
        pl.kernel,
        mesh=mesh,
        out_type=jax.ShapeDtypeStruct((batch, dim), jnp.float32),
        compiler_params=pltpu.CompilerParams(use_tc_tiling_on_sc=False),
        scratch_types=[
            pltpu.VMEM((n_chunks, CHUNK), jnp.int32),
            pltpu.VMEM((NBUF, CHUNK, dim), jnp.float32),
            pltpu.SemaphoreType.DMA((NBUF,)),
            pltpu.SemaphoreType.DMA((NBUF,)),
        ],
    )
    def gather_kernel(idx_hbm, table_hbm, out_hbm, idx_v, bufs, gsem, osem):
        wid = lax.axis_index("s") * NUM_CORES + lax.axis_index("c")
        base = wid * rows_per_worker
        pltpu.sync_copy(idx_hbm.at[wid], idx_v)

        def gather_start(g, slot):
            pltpu.async_copy(table_hbm.at[idx_v.at[g]], bufs.at[slot],
                             gsem.at[slot])

        def gather_wait(g, slot):
            pltpu.make_async_copy(table_hbm.at[idx_v.at[g]], bufs.at[slot],
                                  gsem.at[slot]).wait()

        def out_start(g, slot):
            pltpu.async_copy(bufs.at[slot],
                             out_hbm.at[pl.ds(base + g * CHUNK, CHUNK)],
                             osem.at[slot])

        def out_wait(g, slot):
            pltpu.make_async_copy(bufs.at[slot],
                                  out_hbm.at[pl.ds(base + g * CHUNK, CHUNK)],
                                  osem.at[slot]).wait()

        for s in range(LOOKAHEAD):
            gather_start(s, s)
        for g in range(NBUF - LOOKAHEAD):
            gather_start(g + LOOKAHEAD, g + LOOKAHEAD)
            gather_wait(g, g)
            out_start(g, g)

        start = NBUF - LOOKAHEAD
        n_main = n_chunks - NBUF

        def body(t, carry):
            for s in range(NBUF):
                g = start + t * NBUF + s
                slot = (start + s) % NBUF
                f = g + LOOKAHEAD
                fslot = (slot + LOOKAHEAD) % NBUF
                out_wait(f - NBUF, fslot)
                gather_start(f, fslot)
                gather_wait(g, slot)
                out_start(g, slot)
            return carry

        lax.fori_loop(0, n_main // NBUF, body, 0)

        for k in range(LOOKAHEAD):
            g = n_chunks - LOOKAHEAD + k
            slot = g % NBUF
            gather_wait(g, slot)
            out_start(g, slot)
        for k in range(NBUF):
            g = n_chunks - NBUF + k
            out_wait(g, g % NBUF)

    return gather_kernel


def kernel(input, weight):
    b0, b1 = input.shape
    num_rows, dim = weight.shape
    idx = input.reshape(NUM_WORKERS, -1, CHUNK)
    if idx.dtype != jnp.int32:
        idx = idx.astype(jnp.int32)
    out = _make_gather(num_rows, dim, b0 * b1)(idx, weight)
    return out.reshape(b0, b1, dim)
